# baseline (speedup 1.0000x reference)
import functools

import jax
import jax.numpy as jnp
from jax import lax
from jax.experimental import pallas as pl
from jax.experimental.pallas import tpu as pltpu
from jax.experimental.pallas import tpu_sc as plsc

N = 10000
E = 320000
IN_CH = 128
OUT_CH = 128
HID = 256

NC = 2
NS = 16
NW = NC * NS

CHUNK = 128
CHUNKS_PER_TILE = 80
E_PER_TILE = CHUNK * CHUNKS_PER_TILE
E_PAD = NW * E_PER_TILE
NBUF = 1

N_PAD = 10112
ROWS_PER_TILE = N_PAD // NS

def _sc_degree_body(col_hbm, ones_hbm, z_hbm, out_hbm,
                    col_v, ones_v, acc_sh, sem, ssem):
    cid = lax.axis_index("c")
    sid = lax.axis_index("s")
    wid = cid * NS + sid
    pltpu.sync_copy(z_hbm, acc_sh.at[pl.ds(sid * ROWS_PER_TILE, ROWS_PER_TILE)])
    pltpu.sync_copy(ones_hbm, ones_v)
    pltpu.sync_copy(col_hbm.at[wid], col_v)
    plsc.subcore_barrier()

    @pl.loop(0, CHUNKS_PER_TILE)
    def _(j):
        pltpu.async_copy(ones_v, acc_sh.at[col_v.at[j]], ssem, add=True)

    @pl.loop(0, CHUNKS_PER_TILE)
    def _(j):
        pltpu.make_async_copy(ones_v, acc_sh.at[col_v.at[j]], ssem).wait()

    plsc.subcore_barrier()
    r0 = sid * ROWS_PER_TILE
    pltpu.async_copy(acc_sh.at[pl.ds(r0, ROWS_PER_TILE)],
                     out_hbm.at[cid, pl.ds(r0, ROWS_PER_TILE)], sem).wait()


@functools.cache
def _sc_degree():
    mesh = plsc.VectorSubcoreMesh(core_axis_name="c", subcore_axis_name="s")
    return pl.kernel(
        _sc_degree_body,
        out_type=jax.ShapeDtypeStruct((NC, N_PAD, IN_CH), jnp.float32),
        mesh=mesh,
        scratch_types=[
            pltpu.VMEM((CHUNKS_PER_TILE, CHUNK), jnp.int32),
            pltpu.VMEM((CHUNK, IN_CH), jnp.float32),
            pltpu.VMEM_SHARED((N_PAD, IN_CH), jnp.float32),
            pltpu.SemaphoreType.DMA,
            pltpu.SemaphoreType.DMA,
        ],
    )


HALF = CHUNKS_PER_TILE // 2


def _sc_agg_body(y_hbm, row_hbm, col_hbm, z_hbm, out_hbm,
                 row_v, col_v, b0, b1, g0, g1, s0, s1, sem, acc_sh):
    bufs = (b0, b1)
    gsems = (g0, g1)
    ssems = (s0, s1)
    cid = lax.axis_index("c")
    sid = lax.axis_index("s")
    wid = cid * NS + sid
    pltpu.sync_copy(z_hbm, acc_sh.at[pl.ds(sid * ROWS_PER_TILE, ROWS_PER_TILE)])
    plsc.subcore_barrier()

    for half in range(2):
        pltpu.sync_copy(row_hbm.at[wid, pl.ds(half * HALF, HALF)], row_v)
        pltpu.sync_copy(col_hbm.at[wid, pl.ds(half * HALF, HALF)], col_v)
        for b in range(NBUF):
            pltpu.async_copy(y_hbm.at[row_v.at[b]], bufs[b], gsems[b])

        @pl.loop(0, HALF // NBUF - 1)
        def _(g):
            for b in range(NBUF):
                j = g * NBUF + b
                pltpu.make_async_copy(y_hbm.at[row_v.at[j]], bufs[b],
                                      gsems[b]).wait()
                pltpu.async_copy(bufs[b], acc_sh.at[col_v.at[j]], ssems[b],
                                 add=True)
            for b in range(NBUF):
                j = g * NBUF + b
                pltpu.make_async_copy(bufs[b], acc_sh.at[col_v.at[j]],
                                      ssems[b]).wait()
                pltpu.async_copy(y_hbm.at[row_v.at[j + NBUF]], bufs[b],
                                 gsems[b])

        for b in range(NBUF):
            j = HALF - NBUF + b
            pltpu.make_async_copy(y_hbm.at[row_v.at[j]], bufs[b],
                                  gsems[b]).wait()
            pltpu.async_copy(bufs[b], acc_sh.at[col_v.at[j]], ssems[b],
                             add=True)
        for b in range(NBUF):
            j = HALF - NBUF + b
            pltpu.make_async_copy(bufs[b], acc_sh.at[col_v.at[j]],
                                  ssems[b]).wait()

    plsc.subcore_barrier()
    r0 = sid * ROWS_PER_TILE
    pltpu.async_copy(acc_sh.at[pl.ds(r0, ROWS_PER_TILE)],
                     out_hbm.at[cid, pl.ds(r0, ROWS_PER_TILE)], sem).wait()


@functools.cache
def _sc_agg():
    mesh = plsc.VectorSubcoreMesh(core_axis_name="c", subcore_axis_name="s")
    return pl.kernel(
        _sc_agg_body,
        out_type=jax.ShapeDtypeStruct((NC, N_PAD, IN_CH), jnp.float32),
        mesh=mesh,
        scratch_types=[
            pltpu.VMEM((HALF, CHUNK), jnp.int32),
            pltpu.VMEM((HALF, CHUNK), jnp.int32),
            pltpu.VMEM((CHUNK, IN_CH), jnp.float32),
            pltpu.VMEM((CHUNK, IN_CH), jnp.float32),
            pltpu.SemaphoreType.DMA,
            pltpu.SemaphoreType.DMA,
            pltpu.SemaphoreType.DMA,
            pltpu.SemaphoreType.DMA,
            pltpu.SemaphoreType.DMA,
            pltpu.VMEM_SHARED((N_PAD, IN_CH), jnp.float32),
        ],
    )


BLK = 1000
GRID = N // BLK


def _tc_prep_body(x_ref, dp_ref, y1_ref, dis_ref):
    p = dp_ref[0, :, :16] + dp_ref[1, :, :16] + 1.0
    dis = 1.0 / jnp.sqrt(p)
    x = x_ref[...]
    xf = (x * x) / x
    y1_ref[...] = xf * dis[:, :1]
    dis_ref[...] = dis


def _tc_mid_body(p_ref, y1_ref, dis_ref, w1_ref, b1_ref, w2_ref, y2_ref):
    dis = dis_ref[:, :1]
    agg = (p_ref[0] + p_ref[1] + y1_ref[...]) * dis
    h = jnp.dot(agg, w1_ref[...], precision=lax.Precision.HIGHEST,
                preferred_element_type=jnp.float32) + b1_ref[...]
    h = jnp.maximum(h, 0.0)
    t = jnp.dot(h, w2_ref[...], precision=lax.Precision.HIGHEST,
                preferred_element_type=jnp.float32)
    y2_ref[...] = t * dis


def _tc_final_body(p_ref, y2_ref, dis_ref, b2_ref, out_ref):
    dis = dis_ref[:, :1]
    out_ref[...] = (p_ref[0] + p_ref[1] + y2_ref[...]) * dis + b2_ref[...]


def _row_blocks(width):
    return pl.BlockSpec((BLK, width), lambda i: (i, 0))


def _part_blocks(width):
    return pl.BlockSpec((NC, BLK, width), lambda i: (0, i, 0))


def _full(shape):
    return pl.BlockSpec(shape, lambda i: tuple(0 for _ in shape))


_tc_prep = pl.pallas_call(
    _tc_prep_body,
    grid=(GRID,),
    in_specs=[_row_blocks(IN_CH), _part_blocks(IN_CH)],
    out_specs=[_row_blocks(IN_CH), _row_blocks(16)],
    out_shape=[jax.ShapeDtypeStruct((N, IN_CH), jnp.float32),
               jax.ShapeDtypeStruct((N, 16), jnp.float32)],
)

_tc_mid = pl.pallas_call(
    _tc_mid_body,
    grid=(GRID,),
    in_specs=[_part_blocks(IN_CH), _row_blocks(IN_CH), _row_blocks(16),
              _full((IN_CH, HID)), _full((1, HID)), _full((HID, OUT_CH))],
    out_specs=_row_blocks(OUT_CH),
    out_shape=jax.ShapeDtypeStruct((N, OUT_CH), jnp.float32),
)

_tc_final = pl.pallas_call(
    _tc_final_body,
    grid=(GRID,),
    in_specs=[_part_blocks(OUT_CH), _row_blocks(OUT_CH), _row_blocks(16),
              _full((1, OUT_CH))],
    out_specs=_row_blocks(OUT_CH),
    out_shape=jax.ShapeDtypeStruct((N, OUT_CH), jnp.float32),
)


@jax.jit
def kernel(x, edge_index, W1, b1, W2, b2):
    row = edge_index[0].astype(jnp.int32)
    col = edge_index[1].astype(jnp.int32)
    pad = E_PAD - E
    pad_rows = (jnp.arange(pad, dtype=jnp.int32) * 37) % N
    pad_cols = N + (jnp.arange(pad, dtype=jnp.int32) % (N_PAD - N))
    row_p = jnp.concatenate([row, pad_rows])
    col_p = jnp.concatenate([col, pad_cols])
    row_p = row_p.reshape(NW, CHUNKS_PER_TILE, CHUNK)
    col_p = col_p.reshape(NW, CHUNKS_PER_TILE, CHUNK)

    ones128 = jnp.ones((CHUNK, IN_CH), jnp.float32)
    z128 = jnp.zeros((ROWS_PER_TILE, IN_CH), jnp.float32)

    deg_parts = _sc_degree()(col_p, ones128, z128)
    y1, dis16 = _tc_prep(x, deg_parts)
    p1 = _sc_agg()(y1, row_p, col_p, z128)
    y2 = _tc_mid(p1, y1, dis16, W1, b1.reshape(1, HID), W2)
    p2 = _sc_agg()(y2, row_p, col_p, z128)
    out = _tc_final(p2, y2, dis16, b2.reshape(1, OUT_CH))
    return out


# speedup vs baseline: 1.3004x; 1.3004x over previous
import dataclasses
import functools

import jax
import jax.numpy as jnp
from jax import lax
from jax.experimental import pallas as pl
from jax.experimental.pallas import tpu as pltpu
from jax.experimental.pallas import tpu_sc as plsc

N = 10000
E = 320000
IN_CH = 128
OUT_CH = 128
HID = 256

NC = 2
NS = 16
NW = NC * NS

CHUNK = 128
CHUNKS_PER_TILE = 80
E_PER_TILE = CHUNK * CHUNKS_PER_TILE
E_PAD = NW * E_PER_TILE
NBUF = 2

N_PAD = 10112
ROWS_PER_TILE = N_PAD // NS

RED = 640
RED_LAST = N_PAD - 15 * RED


def _sc_degree_body(col_hbm, out_hbm, col_v, hist_v, red_v, outb_v,
                    stage_sh, sem):
    cid = lax.axis_index("c")
    sid = lax.axis_index("s")
    wid = cid * NS + sid
    pltpu.sync_copy(col_hbm.at[wid], col_v)

    @pl.loop(0, N_PAD // 16)
    def _(i):
        hist_v[pl.ds(i * 16, 16)] = jnp.zeros((16,), jnp.float32)

    ones = jnp.ones((16,), jnp.float32)

    @pl.loop(0, CHUNKS_PER_TILE)
    def _(j):
        for k in range(CHUNK // 16):
            idx = col_v[j, pl.ds(k * 16, 16)]
            plsc.addupdate_scatter(hist_v, [idx], ones)

    pltpu.sync_copy(hist_v, stage_sh.at[sid])
    plsc.subcore_barrier()

    def reduce_range(r0, size):
        pltpu.sync_copy(stage_sh.at[:, pl.ds(r0, size)],
                        red_v.at[:, pl.ds(0, size)])

        @pl.loop(0, size // 16)
        def _(k):
            v = red_v[0, pl.ds(k * 16, 16)]
            for t in range(1, NS):
                v = v + red_v[t, pl.ds(k * 16, 16)]
            for l in range(16):
                lane = lax.gather(
                    v, jnp.full((16, 1), l, jnp.int32),
                    lax.GatherDimensionNumbers(offset_dims=(),
                                               collapsed_slice_dims=(0,),
                                               start_index_map=(0,)),
                    (1,), mode=lax.GatherScatterMode.PROMISE_IN_BOUNDS)
                outb_v[k * 16 + l, :] = lane

        pltpu.sync_copy(outb_v.at[pl.ds(0, size)],
                        out_hbm.at[cid, pl.ds(r0, size)], )

    @pl.when(sid < NS - 1)
    def _():
        reduce_range(sid * RED, RED)

    @pl.when(sid == NS - 1)
    def _():
        reduce_range((NS - 1) * RED, RED_LAST)


@functools.cache
def _sc_degree():
    mesh = plsc.VectorSubcoreMesh(core_axis_name="c", subcore_axis_name="s")
    cp = pltpu.CompilerParams()
    if "needs_layout_passes" in pltpu.CompilerParams.__dataclass_fields__:
        cp = dataclasses.replace(cp, needs_layout_passes=False)
    return pl.kernel(
        _sc_degree_body,
        out_type=jax.ShapeDtypeStruct((NC, N_PAD, 16), jnp.float32),
        mesh=mesh,
        compiler_params=cp,
        scratch_types=[
            pltpu.VMEM((CHUNKS_PER_TILE, CHUNK), jnp.int32),
            pltpu.VMEM((N_PAD,), jnp.float32),
            pltpu.VMEM((NS, RED), jnp.float32),
            pltpu.VMEM((RED, 16), jnp.float32),
            pltpu.VMEM_SHARED((NS, N_PAD), jnp.float32),
            pltpu.SemaphoreType.DMA,
        ],
    )


HALF = CHUNKS_PER_TILE // 2


def _sc_agg_body(y_hbm, row_hbm, col_hbm, z_hbm, out_hbm,
                 row_v, col_v, b0, b1, g0, g1, s0, s1, sem, acc_sh):
    bufs = (b0, b1)
    gsems = (g0, g1)
    ssems = (s0, s1)
    cid = lax.axis_index("c")
    sid = lax.axis_index("s")
    wid = cid * NS + sid
    pltpu.sync_copy(z_hbm, acc_sh.at[pl.ds(sid * ROWS_PER_TILE, ROWS_PER_TILE)])
    plsc.subcore_barrier()

    for half in range(2):
        pltpu.sync_copy(row_hbm.at[wid, pl.ds(half * HALF, HALF)], row_v)
        pltpu.sync_copy(col_hbm.at[wid, pl.ds(half * HALF, HALF)], col_v)
        for b in range(NBUF):
            pltpu.async_copy(y_hbm.at[row_v.at[b]], bufs[b], gsems[b])

        @pl.loop(0, HALF // NBUF - 1)
        def _(g):
            for b in range(NBUF):
                j = g * NBUF + b
                pltpu.make_async_copy(y_hbm.at[row_v.at[j]], bufs[b],
                                      gsems[b]).wait()
                pltpu.async_copy(bufs[b], acc_sh.at[col_v.at[j]], ssems[b],
                                 add=True)
            for b in range(NBUF):
                j = g * NBUF + b
                pltpu.make_async_copy(bufs[b], acc_sh.at[col_v.at[j]],
                                      ssems[b]).wait()
                pltpu.async_copy(y_hbm.at[row_v.at[j + NBUF]], bufs[b],
                                 gsems[b])

        for b in range(NBUF):
            j = HALF - NBUF + b
            pltpu.make_async_copy(y_hbm.at[row_v.at[j]], bufs[b],
                                  gsems[b]).wait()
            pltpu.async_copy(bufs[b], acc_sh.at[col_v.at[j]], ssems[b],
                             add=True)
        for b in range(NBUF):
            j = HALF - NBUF + b
            pltpu.make_async_copy(bufs[b], acc_sh.at[col_v.at[j]],
                                  ssems[b]).wait()

    plsc.subcore_barrier()
    r0 = sid * ROWS_PER_TILE
    pltpu.async_copy(acc_sh.at[pl.ds(r0, ROWS_PER_TILE)],
                     out_hbm.at[cid, pl.ds(r0, ROWS_PER_TILE)], sem).wait()


@functools.cache
def _sc_agg():
    mesh = plsc.VectorSubcoreMesh(core_axis_name="c", subcore_axis_name="s")
    return pl.kernel(
        _sc_agg_body,
        out_type=jax.ShapeDtypeStruct((NC, N_PAD, IN_CH), jnp.float32),
        mesh=mesh,
        scratch_types=[
            pltpu.VMEM((HALF, CHUNK), jnp.int32),
            pltpu.VMEM((HALF, CHUNK), jnp.int32),
            pltpu.VMEM((CHUNK, IN_CH), jnp.float32),
            pltpu.VMEM((CHUNK, IN_CH), jnp.float32),
            pltpu.SemaphoreType.DMA,
            pltpu.SemaphoreType.DMA,
            pltpu.SemaphoreType.DMA,
            pltpu.SemaphoreType.DMA,
            pltpu.SemaphoreType.DMA,
            pltpu.VMEM_SHARED((N_PAD, IN_CH), jnp.float32),
        ],
    )


BLK = 1000
GRID = N // BLK


def _tc_prep_body(x_ref, dp_ref, y1_ref, dis_ref):
    p = dp_ref[0] + dp_ref[1] + 1.0
    dis = 1.0 / jnp.sqrt(p)
    x = x_ref[...]
    xf = (x * x) / x
    y1_ref[...] = xf * dis[:, :1]
    dis_ref[...] = dis


def _tc_mid_body(p_ref, y1_ref, dis_ref, w1_ref, b1_ref, w2_ref, y2_ref):
    dis = dis_ref[:, :1]
    agg = (p_ref[0] + p_ref[1] + y1_ref[...]) * dis
    h = jnp.dot(agg, w1_ref[...], precision=lax.Precision.HIGHEST,
                preferred_element_type=jnp.float32) + b1_ref[...]
    h = jnp.maximum(h, 0.0)
    t = jnp.dot(h, w2_ref[...], precision=lax.Precision.HIGHEST,
                preferred_element_type=jnp.float32)
    y2_ref[...] = t * dis


def _tc_final_body(p_ref, y2_ref, dis_ref, b2_ref, out_ref):
    dis = dis_ref[:, :1]
    out_ref[...] = (p_ref[0] + p_ref[1] + y2_ref[...]) * dis + b2_ref[...]


def _row_blocks(width):
    return pl.BlockSpec((BLK, width), lambda i: (i, 0))


def _part_blocks(width):
    return pl.BlockSpec((NC, BLK, width), lambda i: (0, i, 0))


def _full(shape):
    return pl.BlockSpec(shape, lambda i: tuple(0 for _ in shape))


_tc_prep = pl.pallas_call(
    _tc_prep_body,
    grid=(GRID,),
    in_specs=[_row_blocks(IN_CH), _part_blocks(16)],
    out_specs=[_row_blocks(IN_CH), _row_blocks(16)],
    out_shape=[jax.ShapeDtypeStruct((N, IN_CH), jnp.float32),
               jax.ShapeDtypeStruct((N, 16), jnp.float32)],
)

_tc_mid = pl.pallas_call(
    _tc_mid_body,
    grid=(GRID,),
    in_specs=[_part_blocks(IN_CH), _row_blocks(IN_CH), _row_blocks(16),
              _full((IN_CH, HID)), _full((1, HID)), _full((HID, OUT_CH))],
    out_specs=_row_blocks(OUT_CH),
    out_shape=jax.ShapeDtypeStruct((N, OUT_CH), jnp.float32),
)

_tc_final = pl.pallas_call(
    _tc_final_body,
    grid=(GRID,),
    in_specs=[_part_blocks(OUT_CH), _row_blocks(OUT_CH), _row_blocks(16),
              _full((1, OUT_CH))],
    out_specs=_row_blocks(OUT_CH),
    out_shape=jax.ShapeDtypeStruct((N, OUT_CH), jnp.float32),
)


@jax.jit
def kernel(x, edge_index, W1, b1, W2, b2):
    row = edge_index[0].astype(jnp.int32)
    col = edge_index[1].astype(jnp.int32)
    pad = E_PAD - E
    pad_rows = (jnp.arange(pad, dtype=jnp.int32) * 37) % N
    pad_cols = N + (jnp.arange(pad, dtype=jnp.int32) % (N_PAD - N))
    row_p = jnp.concatenate([row, pad_rows])
    col_p = jnp.concatenate([col, pad_cols])
    row_p = row_p.reshape(NW, CHUNKS_PER_TILE, CHUNK)
    col_p = col_p.reshape(NW, CHUNKS_PER_TILE, CHUNK)

    z128 = jnp.zeros((ROWS_PER_TILE, IN_CH), jnp.float32)

    deg_parts = _sc_degree()(col_p)
    y1, dis16 = _tc_prep(x, deg_parts)
    p1 = _sc_agg()(y1, row_p, col_p, z128)
    y2 = _tc_mid(p1, y1, dis16, W1, b1.reshape(1, HID), W2)
    p2 = _sc_agg()(y2, row_p, col_p, z128)
    out = _tc_final(p2, y2, dis16, b2.reshape(1, OUT_CH))
    return out


# speedup vs baseline: 1.4073x; 1.0822x over previous
import dataclasses
import functools

import jax
import jax.numpy as jnp
import numpy as np
from jax import lax
from jax.experimental import pallas as pl
from jax.experimental.pallas import tpu as pltpu
from jax.experimental.pallas import tpu_sc as plsc

N = 10000
E = 320000
IN_CH = 128
OUT_CH = 128
HID = 256

NC = 2
NS = 16
NW = NC * NS

CHUNK = 128
CHUNKS_PER_TILE = 80
E_PER_TILE = CHUNK * CHUNKS_PER_TILE
E_PAD = NW * E_PER_TILE
NBUF = 2

N_PAD = 10112
ROWS_PER_TILE = N_PAD // NS

RED = 640
RED_LAST = N_PAD - 15 * RED


def _sc_degree_body(col_hbm, out_hbm, col_v, hist_v, red_v, outb_v,
                    stage_sh, sem):
    cid = lax.axis_index("c")
    sid = lax.axis_index("s")
    wid = cid * NS + sid
    pltpu.sync_copy(col_hbm.at[wid], col_v)

    @pl.loop(0, N_PAD // 16)
    def _(i):
        hist_v[pl.ds(i * 16, 16)] = jnp.zeros((16,), jnp.float32)

    ones = jnp.ones((16,), jnp.float32)

    @pl.loop(0, CHUNKS_PER_TILE)
    def _(j):
        for k in range(CHUNK // 16):
            idx = col_v[j, pl.ds(k * 16, 16)]
            plsc.addupdate_scatter(hist_v, [idx], ones)

    pltpu.sync_copy(hist_v, stage_sh.at[sid])
    plsc.subcore_barrier()

    def reduce_range(r0, size):
        pltpu.sync_copy(stage_sh.at[:, pl.ds(r0, size)],
                        red_v.at[:, pl.ds(0, size)])

        @pl.loop(0, size // 16)
        def _(k):
            v = red_v[0, pl.ds(k * 16, 16)]
            for t in range(1, NS):
                v = v + red_v[t, pl.ds(k * 16, 16)]
            for l in range(16):
                lane = lax.gather(
                    v, jnp.full((16, 1), l, jnp.int32),
                    lax.GatherDimensionNumbers(offset_dims=(),
                                               collapsed_slice_dims=(0,),
                                               start_index_map=(0,)),
                    (1,), mode=lax.GatherScatterMode.PROMISE_IN_BOUNDS)
                outb_v[k * 16 + l, :] = lane

        pltpu.sync_copy(outb_v.at[pl.ds(0, size)],
                        out_hbm.at[cid, pl.ds(r0, size)], )

    @pl.when(sid < NS - 1)
    def _():
        reduce_range(sid * RED, RED)

    @pl.when(sid == NS - 1)
    def _():
        reduce_range((NS - 1) * RED, RED_LAST)


@functools.cache
def _sc_degree():
    mesh = plsc.VectorSubcoreMesh(core_axis_name="c", subcore_axis_name="s")
    cp = pltpu.CompilerParams()
    if "needs_layout_passes" in pltpu.CompilerParams.__dataclass_fields__:
        cp = dataclasses.replace(cp, needs_layout_passes=False)
    return pl.kernel(
        _sc_degree_body,
        out_type=jax.ShapeDtypeStruct((NC, N_PAD, 16), jnp.float32),
        mesh=mesh,
        compiler_params=cp,
        scratch_types=[
            pltpu.VMEM((CHUNKS_PER_TILE, CHUNK), jnp.int32),
            pltpu.VMEM((N_PAD,), jnp.float32),
            pltpu.VMEM((NS, RED), jnp.float32),
            pltpu.VMEM((RED, 16), jnp.float32),
            pltpu.VMEM_SHARED((NS, N_PAD), jnp.float32),
            pltpu.SemaphoreType.DMA,
        ],
    )


HALF = CHUNKS_PER_TILE // 2


def _sc_agg_body(y_hbm, row_hbm, col_hbm, z_hbm, out_hbm,
                 row_v, col_v, b0, b1, g0, g1, s0, s1, sem, acc_sh):
    bufs = (b0, b1)
    gsems = (g0, g1)
    ssems = (s0, s1)
    cid = lax.axis_index("c")
    sid = lax.axis_index("s")
    wid = cid * NS + sid
    pltpu.sync_copy(z_hbm, acc_sh.at[pl.ds(sid * ROWS_PER_TILE, ROWS_PER_TILE)])
    plsc.subcore_barrier()

    for half in range(2):
        pltpu.sync_copy(row_hbm.at[wid, pl.ds(half * HALF, HALF)], row_v)
        pltpu.sync_copy(col_hbm.at[wid, pl.ds(half * HALF, HALF)], col_v)
        for b in range(NBUF):
            pltpu.async_copy(y_hbm.at[row_v.at[b]], bufs[b], gsems[b])

        @pl.loop(0, HALF // NBUF - 1)
        def _(g):
            for b in range(NBUF):
                j = g * NBUF + b
                pltpu.make_async_copy(y_hbm.at[row_v.at[j]], bufs[b],
                                      gsems[b]).wait()
                pltpu.async_copy(bufs[b], acc_sh.at[col_v.at[j]], ssems[b],
                                 add=True)
            for b in range(NBUF):
                j = g * NBUF + b
                pltpu.make_async_copy(bufs[b], acc_sh.at[col_v.at[j]],
                                      ssems[b]).wait()
                pltpu.async_copy(y_hbm.at[row_v.at[j + NBUF]], bufs[b],
                                 gsems[b])

        for b in range(NBUF):
            j = HALF - NBUF + b
            pltpu.make_async_copy(y_hbm.at[row_v.at[j]], bufs[b],
                                  gsems[b]).wait()
            pltpu.async_copy(bufs[b], acc_sh.at[col_v.at[j]], ssems[b],
                             add=True)
        for b in range(NBUF):
            j = HALF - NBUF + b
            pltpu.make_async_copy(bufs[b], acc_sh.at[col_v.at[j]],
                                  ssems[b]).wait()

    plsc.subcore_barrier()
    r0 = sid * ROWS_PER_TILE
    pltpu.async_copy(acc_sh.at[pl.ds(r0, ROWS_PER_TILE)],
                     out_hbm.at[cid, pl.ds(r0, ROWS_PER_TILE)], sem).wait()


@functools.cache
def _sc_agg():
    mesh = plsc.VectorSubcoreMesh(core_axis_name="c", subcore_axis_name="s")
    return pl.kernel(
        _sc_agg_body,
        out_type=jax.ShapeDtypeStruct((NC, N_PAD, IN_CH), jnp.float32),
        mesh=mesh,
        scratch_types=[
            pltpu.VMEM((HALF, CHUNK), jnp.int32),
            pltpu.VMEM((HALF, CHUNK), jnp.int32),
            pltpu.VMEM((CHUNK, IN_CH), jnp.float32),
            pltpu.VMEM((CHUNK, IN_CH), jnp.float32),
            pltpu.SemaphoreType.DMA,
            pltpu.SemaphoreType.DMA,
            pltpu.SemaphoreType.DMA,
            pltpu.SemaphoreType.DMA,
            pltpu.SemaphoreType.DMA,
            pltpu.VMEM_SHARED((N_PAD, IN_CH), jnp.float32),
        ],
    )


BLK = 2000
GRID = N // BLK


def _tc_prep_body(x_ref, dp_ref, y1_ref, dis_ref):
    p = dp_ref[0] + dp_ref[1] + 1.0
    dis = 1.0 / jnp.sqrt(p)
    x = x_ref[...]
    xf = (x * x) / x
    y1_ref[...] = xf * dis[:, :1]
    dis_ref[...] = dis


def _tc_mid_body(p_ref, y1_ref, dis_ref, w1_ref, b1_ref, w2_ref, y2_ref):
    dis = dis_ref[:, :1]
    agg = (p_ref[0] + p_ref[1] + y1_ref[...]) * dis
    h = jnp.dot(agg.astype(jnp.bfloat16), w1_ref[...].astype(jnp.bfloat16),
                preferred_element_type=jnp.float32) + b1_ref[...]
    h = jnp.maximum(h, 0.0)
    t = jnp.dot(h.astype(jnp.bfloat16), w2_ref[...].astype(jnp.bfloat16),
                preferred_element_type=jnp.float32)
    y2_ref[...] = t * dis


def _tc_final_body(p_ref, y2_ref, dis_ref, b2_ref, out_ref):
    dis = dis_ref[:, :1]
    out_ref[...] = (p_ref[0] + p_ref[1] + y2_ref[...]) * dis + b2_ref[...]


def _row_blocks(width):
    return pl.BlockSpec((BLK, width), lambda i: (i, 0))


def _part_blocks(width):
    return pl.BlockSpec((NC, BLK, width), lambda i: (0, i, 0))


def _full(shape):
    return pl.BlockSpec(shape, lambda i: tuple(0 for _ in shape))


_tc_prep = pl.pallas_call(
    _tc_prep_body,
    grid=(GRID,),
    in_specs=[_row_blocks(IN_CH), _part_blocks(16)],
    out_specs=[_row_blocks(IN_CH), _row_blocks(16)],
    out_shape=[jax.ShapeDtypeStruct((N, IN_CH), jnp.float32),
               jax.ShapeDtypeStruct((N, 16), jnp.float32)],
)

_tc_mid = pl.pallas_call(
    _tc_mid_body,
    grid=(GRID,),
    in_specs=[_part_blocks(IN_CH), _row_blocks(IN_CH), _row_blocks(16),
              _full((IN_CH, HID)), _full((1, HID)), _full((HID, OUT_CH))],
    out_specs=_row_blocks(OUT_CH),
    out_shape=jax.ShapeDtypeStruct((N, OUT_CH), jnp.float32),
)

_tc_final = pl.pallas_call(
    _tc_final_body,
    grid=(GRID,),
    in_specs=[_part_blocks(OUT_CH), _row_blocks(OUT_CH), _row_blocks(16),
              _full((1, OUT_CH))],
    out_specs=_row_blocks(OUT_CH),
    out_shape=jax.ShapeDtypeStruct((N, OUT_CH), jnp.float32),
)


@jax.jit
def kernel(x, edge_index, W1, b1, W2, b2):
    row = edge_index[0].astype(jnp.int32)
    col = edge_index[1].astype(jnp.int32)
    pad = E_PAD - E
    pad_rows = jnp.asarray((np.arange(pad) * 37) % N, jnp.int32)
    pad_cols = jnp.asarray(N + np.arange(pad) % (N_PAD - N), jnp.int32)
    row_p = jnp.concatenate([row, pad_rows])
    col_p = jnp.concatenate([col, pad_cols])
    row_p = row_p.reshape(NW, CHUNKS_PER_TILE, CHUNK)
    col_p = col_p.reshape(NW, CHUNKS_PER_TILE, CHUNK)

    z128 = jnp.zeros((ROWS_PER_TILE, IN_CH), jnp.float32)

    deg_parts = _sc_degree()(col_p)
    y1, dis16 = _tc_prep(x, deg_parts)
    p1 = _sc_agg()(y1, row_p, col_p, z128)
    y2 = _tc_mid(p1, y1, dis16, W1, b1.reshape(1, HID), W2)
    p2 = _sc_agg()(y2, row_p, col_p, z128)
    out = _tc_final(p2, y2, dis16, b2.reshape(1, OUT_CH))
    return out
